# Initial kernel scaffold; baseline (speedup 1.0000x reference)
import jax
import jax.numpy as jnp
from jax.experimental import pallas as pl


def kernel(x, pe_table):
    raise NotImplementedError("write your pallas kernel here")


# speedup vs baseline: 1.9263x; 1.9263x over previous
import functools
import math

import jax
import jax.numpy as jnp
from jax.experimental import pallas as pl


def _add_pe_block(x_ref, pe_ref, o_ref, *, scale):
    o_ref[...] = x_ref[...] + (pe_ref[...] * scale)[None, :, :]


def kernel(x, pe_table):
    B, S, D = x.shape
    scale = math.sqrt(pe_table.shape[1])
    BS = 256
    assert S % BS == 0
    out = pl.pallas_call(
        functools.partial(_add_pe_block, scale=scale),
        grid=(S // BS,),
        in_specs=[
            pl.BlockSpec((B, BS, D), lambda i: (0, i, 0)),
            pl.BlockSpec((BS, D), lambda i: (i, 0)),
        ],
        out_specs=pl.BlockSpec((B, BS, D), lambda i: (0, i, 0)),
        out_shape=jax.ShapeDtypeStruct((B, S, D), x.dtype),
    )(x, pe_table)
    return out


# speedup vs baseline: 1.9616x; 1.0183x over previous
import functools
import math

import jax
import jax.numpy as jnp
from jax.experimental import pallas as pl


def _add_pe_block(x_ref, pe_ref, o_ref, *, scale):
    o_ref[...] = x_ref[...] + (pe_ref[...] * scale)[None, :, :]


def kernel(x, pe_table):
    B, S, D = x.shape
    scale = math.sqrt(pe_table.shape[1])
    BS = 512
    assert S % BS == 0
    out = pl.pallas_call(
        functools.partial(_add_pe_block, scale=scale),
        grid=(S // BS,),
        in_specs=[
            pl.BlockSpec((B, BS, D), lambda i: (0, i, 0)),
            pl.BlockSpec((BS, D), lambda i: (i, 0)),
        ],
        out_specs=pl.BlockSpec((B, BS, D), lambda i: (0, i, 0)),
        out_shape=jax.ShapeDtypeStruct((B, S, D), x.dtype),
    )(x, pe_table)
    return out


# speedup vs baseline: 1.9634x; 1.0009x over previous
import functools
import math

import jax
import jax.numpy as jnp
from jax.experimental import pallas as pl
from jax.experimental.pallas import tpu as pltpu


def _add_pe_block(x_ref, pe_ref, o_ref, *, scale):
    o_ref[...] = x_ref[...] + (pe_ref[...] * scale)[None, :, :]


def kernel(x, pe_table):
    B, S, D = x.shape
    scale = math.sqrt(pe_table.shape[1])
    BS = 512
    assert S % BS == 0
    out = pl.pallas_call(
        functools.partial(_add_pe_block, scale=scale),
        grid=(S // BS,),
        in_specs=[
            pl.BlockSpec((B, BS, D), lambda i: (0, i, 0)),
            pl.BlockSpec((BS, D), lambda i: (i, 0)),
        ],
        out_specs=pl.BlockSpec((B, BS, D), lambda i: (0, i, 0)),
        out_shape=jax.ShapeDtypeStruct((B, S, D), x.dtype),
        compiler_params=pltpu.CompilerParams(
            dimension_semantics=("parallel",)
        ),
    )(x, pe_table)
    return out
